# baseline (speedup 1.0000x reference)
import functools

import jax
import jax.numpy as jnp
from jax import lax
from jax.experimental import pallas as pl
from jax.experimental.pallas import tpu as pltpu
from jax.experimental.pallas import tpu_sc as plsc

B = 16
MAX_LEN = 4096
TOTAL = 32768
D = 256

CHUNK = 1024
HALF = MAX_LEN // 2
STAGE = 128
LANES = 16


def _body(flat_hbm, cu_hbm, out_hbm, cu_vmem, zbuf, sem):
    c = lax.axis_index("c")
    s = lax.axis_index("s")
    b = s
    p0 = c * HALF

    pltpu.sync_copy(cu_hbm, cu_vmem.at[pl.ds(0, B + 1)])
    cu_pair = cu_vmem[pl.ds(b, LANES)]
    start = pl.multiple_of(cu_pair[0], CHUNK)
    seq_len = cu_pair[1] - cu_pair[0]

    zeros16 = jnp.zeros((LANES,), jnp.float32)

    def _zrow(r, carry):
        for l in range(D // LANES):
            zbuf[r, pl.ds(l * LANES, LANES)] = zeros16
        return carry

    lax.fori_loop(0, STAGE, _zrow, 0)

    for j in range(HALF // CHUNK):
        p = pl.multiple_of(p0 + j * CHUNK, CHUNK)
        valid = (p + CHUNK) <= seq_len

        @pl.when(valid)
        def _copy():
            pltpu.async_copy(
                flat_hbm.at[pl.ds(start + p, CHUNK), :],
                out_hbm.at[b, pl.ds(p, CHUNK), :],
                sem,
            ).wait()

        @pl.when(jnp.logical_not(valid))
        def _pad():
            for t in range(CHUNK // STAGE):
                pltpu.sync_copy(
                    zbuf,
                    out_hbm.at[b, pl.ds(p + t * STAGE, STAGE), :],
                )


_todense = functools.partial(
    pl.kernel,
    out_type=jax.ShapeDtypeStruct((B, MAX_LEN, D), jnp.float32),
    mesh=plsc.VectorSubcoreMesh(core_axis_name="c", subcore_axis_name="s"),
    scratch_types=[
        pltpu.VMEM((2 * LANES,), jnp.int32),
        pltpu.VMEM((STAGE, D), jnp.float32),
        pltpu.SemaphoreType.DMA,
    ],
)(_body)


@jax.jit
def kernel(flat, cu_seqlens):
    return _todense(flat, cu_seqlens)


# speedup vs baseline: 17.9140x; 17.9140x over previous
import functools

import jax
import jax.numpy as jnp
from jax import lax
from jax.experimental import pallas as pl
from jax.experimental.pallas import tpu as pltpu
from jax.experimental.pallas import tpu_sc as plsc

B = 16
MAX_LEN = 4096
TOTAL = 32768
D = 256

HALF = MAX_LEN // 2
STAGE = 128
NSTAGES = HALF // STAGE
LANES = 16


def _body(flat_hbm, cu_hbm, out_hbm, cu_vmem, bufs, zbuf, gsems, psem):
    c = lax.axis_index("c")
    s = lax.axis_index("s")
    b = s
    p0 = c * HALF

    pltpu.sync_copy(cu_hbm, cu_vmem.at[pl.ds(0, B + 1)])
    cu_pair = cu_vmem[pl.ds(b, LANES)]
    start = pl.multiple_of(cu_pair[0], STAGE)
    seq_len = cu_pair[1] - cu_pair[0]

    nvalid = jnp.maximum(0, jnp.minimum(seq_len - p0, HALF)) // STAGE

    def src_at(j):
        return flat_hbm.at[pl.ds(start + (p0 + j * STAGE), STAGE), :]

    def dst_at(j):
        return out_hbm.at[b, pl.ds(p0 + j * STAGE, STAGE), :]

    @pl.when(0 < nvalid)
    def _g0():
        pltpu.async_copy(src_at(0), bufs.at[0], gsems.at[0])

    zeros16 = jnp.zeros((LANES,), jnp.float32)

    def _zrow(r, carry):
        for l in range(D // LANES):
            zbuf[r, pl.ds(l * LANES, LANES)] = zeros16
        return carry

    lax.fori_loop(0, STAGE, _zrow, 0)

    for j in range(NSTAGES):
        cur = j % 2
        nxt = (j + 1) % 2

        @pl.when(j < nvalid)
        def _valid():
            if j + 1 < NSTAGES:
                @pl.when(j + 1 < nvalid)
                def _prefetch():
                    pltpu.async_copy(src_at(j + 1), bufs.at[nxt], gsems.at[nxt])

            pltpu.make_async_copy(src_at(j), bufs.at[cur], gsems.at[cur]).wait()
            pltpu.sync_copy(bufs.at[cur], dst_at(j))

        @pl.when(j >= nvalid)
        def _pad():
            pltpu.async_copy(zbuf, dst_at(j), psem)

    for j in range(NSTAGES):
        @pl.when(j >= nvalid)
        def _drain():
            pltpu.make_async_copy(zbuf, dst_at(j), psem).wait()


_todense = functools.partial(
    pl.kernel,
    out_type=jax.ShapeDtypeStruct((B, MAX_LEN, D), jnp.float32),
    mesh=plsc.VectorSubcoreMesh(core_axis_name="c", subcore_axis_name="s"),
    scratch_types=[
        pltpu.VMEM((2 * LANES,), jnp.int32),
        pltpu.VMEM((2, STAGE, D), jnp.float32),
        pltpu.VMEM((STAGE, D), jnp.float32),
        pltpu.SemaphoreType.DMA((2,)),
        pltpu.SemaphoreType.DMA,
    ],
)(_body)


@jax.jit
def kernel(flat, cu_seqlens):
    return _todense(flat, cu_seqlens)


# speedup vs baseline: 18.2314x; 1.0177x over previous
import functools

import jax
import jax.numpy as jnp
from jax import lax
from jax.experimental import pallas as pl
from jax.experimental.pallas import tpu as pltpu
from jax.experimental.pallas import tpu_sc as plsc

B = 16
MAX_LEN = 4096
TOTAL = 32768
D = 256

HALF = MAX_LEN // 2
STAGE = 128
NSTAGES = HALF // STAGE
LANES = 16


def _body(flat_hbm, cu_hbm, out_hbm, cu_vmem, bufs, zbuf, gsems, psem):
    c = lax.axis_index("c")
    s = lax.axis_index("s")
    b = s
    p0 = c * HALF

    pltpu.sync_copy(cu_hbm, cu_vmem.at[pl.ds(0, B + 1)])
    cu_pair = cu_vmem[pl.ds(b, LANES)]
    start = pl.multiple_of(cu_pair[0], STAGE)
    seq_len = cu_pair[1] - cu_pair[0]

    nvalid = jnp.maximum(0, jnp.minimum(seq_len - p0, HALF)) // STAGE

    def src_at(j):
        off = pl.multiple_of(start + p0 + j * STAGE, STAGE)
        return flat_hbm.at[pl.ds(off, STAGE), :]

    def dst_at(j):
        off = pl.multiple_of(p0 + j * STAGE, STAGE)
        return out_hbm.at[b, pl.ds(off, STAGE), :]

    @pl.when(0 < nvalid)
    def _g0():
        pltpu.async_copy(src_at(0), bufs.at[0], gsems.at[0])

    zeros16 = jnp.zeros((LANES,), jnp.float32)

    def _zrow(r, carry):
        for l in range(D // LANES):
            zbuf[r, pl.ds(l * LANES, LANES)] = zeros16
        return carry

    lax.fori_loop(0, STAGE, _zrow, 0)

    def _valid_stage(j, carry):
        cur = j % 2
        nxt = (j + 1) % 2

        @pl.when(j + 1 < nvalid)
        def _prefetch():
            pltpu.async_copy(src_at(j + 1), bufs.at[nxt], gsems.at[nxt])

        pltpu.make_async_copy(src_at(j), bufs.at[cur], gsems.at[cur]).wait()
        pltpu.sync_copy(bufs.at[cur], dst_at(j))
        return carry

    lax.fori_loop(0, nvalid, _valid_stage, 0)

    def _pad_stage(j, carry):
        pltpu.async_copy(zbuf, dst_at(j), psem)
        return carry

    lax.fori_loop(nvalid, NSTAGES, _pad_stage, 0)

    def _drain_stage(j, carry):
        pltpu.make_async_copy(zbuf, dst_at(j), psem).wait()
        return carry

    lax.fori_loop(nvalid, NSTAGES, _drain_stage, 0)


_todense = functools.partial(
    pl.kernel,
    out_type=jax.ShapeDtypeStruct((B, MAX_LEN, D), jnp.float32),
    mesh=plsc.VectorSubcoreMesh(core_axis_name="c", subcore_axis_name="s"),
    scratch_types=[
        pltpu.VMEM((2 * LANES,), jnp.int32),
        pltpu.VMEM((2, STAGE, D), jnp.float32),
        pltpu.VMEM((STAGE, D), jnp.float32),
        pltpu.SemaphoreType.DMA((2,)),
        pltpu.SemaphoreType.DMA,
    ],
)(_body)


@jax.jit
def kernel(flat, cu_seqlens):
    return _todense(flat, cu_seqlens)


# speedup vs baseline: 18.9361x; 1.0387x over previous
import functools

import jax
import jax.numpy as jnp
from jax import lax
from jax.experimental import pallas as pl
from jax.experimental.pallas import tpu as pltpu
from jax.experimental.pallas import tpu_sc as plsc

B = 16
MAX_LEN = 4096
TOTAL = 32768
D = 256

HALF = MAX_LEN // 2
STAGE = 128
NSTAGES = HALF // STAGE
LANES = 16


def _body(flat_hbm, cu_hbm, out_hbm, cu_vmem, bufs, zbuf, gsems, psem):
    c = lax.axis_index("c")
    s = lax.axis_index("s")
    b = s

    pltpu.sync_copy(cu_hbm, cu_vmem.at[pl.ds(0, B + 1)])
    cu_pair = cu_vmem[pl.ds(b, LANES)]
    start = pl.multiple_of(cu_pair[0], STAGE)
    seq_len = cu_pair[1] - cu_pair[0]

    nvalid = jnp.clip((seq_len - c * STAGE + STAGE) // (2 * STAGE), 0, NSTAGES)

    def src_at(j):
        off = pl.multiple_of(start + j * 2 * STAGE + c * STAGE, STAGE)
        return flat_hbm.at[pl.ds(off, STAGE), :]

    def dst_at(j):
        off = pl.multiple_of(j * 2 * STAGE + c * STAGE, STAGE)
        return out_hbm.at[b, pl.ds(off, STAGE), :]

    @pl.when(0 < nvalid)
    def _g0():
        pltpu.async_copy(src_at(0), bufs.at[0], gsems.at[0])

    zeros16 = jnp.zeros((LANES,), jnp.float32)

    def _zrow(r, carry):
        for l in range(D // LANES):
            zbuf[r, pl.ds(l * LANES, LANES)] = zeros16
        return carry

    lax.fori_loop(0, STAGE, _zrow, 0)

    def _valid_stage(j, carry):
        cur = j % 2
        nxt = (j + 1) % 2

        @pl.when(j + 1 < nvalid)
        def _prefetch():
            pltpu.async_copy(src_at(j + 1), bufs.at[nxt], gsems.at[nxt])

        pltpu.make_async_copy(src_at(j), bufs.at[cur], gsems.at[cur]).wait()
        pltpu.sync_copy(bufs.at[cur], dst_at(j))
        return carry

    lax.fori_loop(0, nvalid, _valid_stage, 0)

    def _pad_stage(j, carry):
        pltpu.async_copy(zbuf, dst_at(j), psem)
        return carry

    lax.fori_loop(nvalid, NSTAGES, _pad_stage, 0)

    def _drain_stage(j, carry):
        pltpu.make_async_copy(zbuf, dst_at(j), psem).wait()
        return carry

    lax.fori_loop(nvalid, NSTAGES, _drain_stage, 0)


_todense = functools.partial(
    pl.kernel,
    out_type=jax.ShapeDtypeStruct((B, MAX_LEN, D), jnp.float32),
    mesh=plsc.VectorSubcoreMesh(core_axis_name="c", subcore_axis_name="s"),
    scratch_types=[
        pltpu.VMEM((2 * LANES,), jnp.int32),
        pltpu.VMEM((2, STAGE, D), jnp.float32),
        pltpu.VMEM((STAGE, D), jnp.float32),
        pltpu.SemaphoreType.DMA((2,)),
        pltpu.SemaphoreType.DMA,
    ],
)(_body)


@jax.jit
def kernel(flat, cu_seqlens):
    return _todense(flat, cu_seqlens)
